# baseline (speedup 1.0000x reference)
import functools

import numpy as np
import jax
import jax.numpy as jnp
from jax import lax
from jax.experimental import pallas as pl
from jax.experimental.pallas import tpu as pltpu
from jax.experimental.pallas import tpu_sc as plsc

L, B, D = 200, 256, 64
LB = L * B
NUM_VAL_TOKENS = 1000

NC, NS = 2, 16
NW = NC * NS
ROWS_PER_W = -(-L // NW)


def _pe_np():
    pos = np.arange(L, dtype=np.float32)[:, None]
    div = np.exp(np.arange(0, D, 2, dtype=np.float32) * (-np.log(10000.0) / D))
    pe = np.zeros((L, D), dtype=np.float32)
    pe[:, 0::2] = np.sin(pos * div)
    pe[:, 1::2] = np.cos(pos * div)
    return pe


_PE = _pe_np()


_sc_mesh = plsc.VectorSubcoreMesh(core_axis_name="c", subcore_axis_name="s")


@functools.partial(
    pl.kernel,
    mesh=_sc_mesh,
    compiler_params=pltpu.CompilerParams(use_tc_tiling_on_sc=False),
    out_type=jax.ShapeDtypeStruct((2 * LB, D), jnp.float32),
    scratch_types=[
        pltpu.VMEM((2, 128), jnp.int32),
        pltpu.VMEM((2, 128), jnp.int32),
        pltpu.VMEM((B, D), jnp.float32),
        pltpu.VMEM((B, D), jnp.float32),
        pltpu.VMEM((L, D), jnp.float32),
        pltpu.SemaphoreType.DMA,
    ],
)
def _sc_gather(nidx_hbm, eidx_hbm, ntab_hbm, etab_hbm, pe_hbm, out_hbm,
               nidx_v, eidx_v, nrows_v, erows_v, pe_v, sem):
    wid = lax.axis_index("s") * NC + lax.axis_index("c")
    pltpu.sync_copy(pe_hbm, pe_v)
    for k in range(ROWS_PER_W):
        l = wid + NW * k

        @pl.when(l < L)
        def _():
            base = l * B
            pltpu.sync_copy(nidx_hbm.at[l], nidx_v)
            pltpu.sync_copy(eidx_hbm.at[l], eidx_v)
            cps = []
            for h in range(2):
                cps.append(pltpu.async_copy(
                    ntab_hbm.at[nidx_v.at[h]],
                    nrows_v.at[pl.ds(h * 128, 128)], sem))
                cps.append(pltpu.async_copy(
                    etab_hbm.at[eidx_v.at[h]],
                    erows_v.at[pl.ds(h * 128, 128)], sem))
            for cp in cps:
                cp.wait()

            p0 = pe_v[l, pl.ds(0, 16)]
            p1 = pe_v[l, pl.ds(16, 16)]
            p2 = pe_v[l, pl.ds(32, 16)]
            p3 = pe_v[l, pl.ds(48, 16)]

            def add_pe(i, c):
                plsc.addupdate(nrows_v.at[i, pl.ds(0, 16)], p0)
                plsc.addupdate(nrows_v.at[i, pl.ds(16, 16)], p1)
                plsc.addupdate(nrows_v.at[i, pl.ds(32, 16)], p2)
                plsc.addupdate(nrows_v.at[i, pl.ds(48, 16)], p3)
                plsc.addupdate(erows_v.at[i, pl.ds(0, 16)], p0)
                plsc.addupdate(erows_v.at[i, pl.ds(16, 16)], p1)
                plsc.addupdate(erows_v.at[i, pl.ds(32, 16)], p2)
                plsc.addupdate(erows_v.at[i, pl.ds(48, 16)], p3)
                return c

            lax.fori_loop(0, B, add_pe, 0)

            pltpu.sync_copy(nrows_v, out_hbm.at[pl.ds(base, B)])
            pltpu.sync_copy(erows_v, out_hbm.at[pl.ds(LB + base, B)])


_BR = 512
_LR = _BR // B


def _tc_body(vmt_ref, wt_ref, pe_ref, out_ref):
    x = jax.lax.dot_general(
        wt_ref[...], vmt_ref[...],
        dimension_numbers=(((1,), (0,)), ((), ())),
        preferred_element_type=jnp.float32)
    for j in range(_LR):
        out_ref[j] = x[:, j * B:(j + 1) * B] + pe_ref[j]


def _tc_matmul(vm_t, w_t, pe3):
    return pl.pallas_call(
        _tc_body,
        grid=(LB // _BR,),
        in_specs=[
            pl.BlockSpec((NUM_VAL_TOKENS, _BR), lambda i: (0, i)),
            pl.BlockSpec((D, NUM_VAL_TOKENS), lambda i: (0, 0)),
            pl.BlockSpec((_LR, D, 1), lambda i: (i, 0, 0)),
        ],
        out_specs=pl.BlockSpec((_LR, D, B), lambda i: (2 * L // _LR + i, 0, 0)),
        out_shape=jax.ShapeDtypeStruct((3 * L, D, B), jnp.float32),
    )(vm_t, w_t, pe3)


def kernel(node_idx, edge_idx, node_val_mat, node_embed_table, edge_embed_table,
           val_tok_embed):
    pe = jnp.asarray(_PE)
    pe3 = jnp.asarray(_PE.reshape(L, D, 1))
    nidx = node_idx.astype(jnp.int32).reshape(L, 2, 128)
    eidx = edge_idx.astype(jnp.int32).reshape(L, 2, 128)
    out3 = _tc_matmul(node_val_mat.T, val_tok_embed.T, pe3)
    sc_out = _sc_gather(nidx, eidx, node_embed_table, edge_embed_table, pe)
    sc3 = sc_out.reshape(2 * L, B, D).transpose(0, 2, 1)
    out3 = jax.lax.dynamic_update_slice(out3, sc3, (0, 0, 0))
    return out3.transpose(0, 2, 1)


# speedup vs baseline: 1.1671x; 1.1671x over previous
import functools

import numpy as np
import jax
import jax.numpy as jnp
from jax import lax
from jax.experimental import pallas as pl
from jax.experimental.pallas import tpu as pltpu
from jax.experimental.pallas import tpu_sc as plsc

L, B, D = 200, 256, 64
LB = L * B
NUM_VAL_TOKENS = 1000

NC, NS = 2, 16
NW = NC * NS
ROWS_PER_W = -(-L // NW)


def _pe_np():
    pos = np.arange(L, dtype=np.float32)[:, None]
    div = np.exp(np.arange(0, D, 2, dtype=np.float32) * (-np.log(10000.0) / D))
    pe = np.zeros((L, D), dtype=np.float32)
    pe[:, 0::2] = np.sin(pos * div)
    pe[:, 1::2] = np.cos(pos * div)
    return pe


_PE = _pe_np()


_sc_mesh = plsc.VectorSubcoreMesh(core_axis_name="c", subcore_axis_name="s")


@functools.partial(
    pl.kernel,
    mesh=_sc_mesh,
    compiler_params=pltpu.CompilerParams(use_tc_tiling_on_sc=False),
    out_type=jax.ShapeDtypeStruct((2 * LB, D), jnp.float32),
    scratch_types=[
        pltpu.VMEM((2, 128), jnp.int32),
        pltpu.VMEM((2, 128), jnp.int32),
        pltpu.VMEM((B, D), jnp.float32),
        pltpu.VMEM((B, D), jnp.float32),
        pltpu.SemaphoreType.DMA,
    ],
)
def _sc_gather(nidx_hbm, eidx_hbm, ntab_hbm, etab_hbm, out_hbm,
               nidx_v, eidx_v, nrows_v, erows_v, sem):
    wid = lax.axis_index("s") * NC + lax.axis_index("c")
    for k in range(ROWS_PER_W):
        l = wid + NW * k

        @pl.when(l < L)
        def _():
            base = l * B
            pltpu.sync_copy(nidx_hbm.at[l], nidx_v)
            pltpu.sync_copy(eidx_hbm.at[l], eidx_v)
            cps = []
            for h in range(2):
                cps.append(pltpu.async_copy(
                    ntab_hbm.at[nidx_v.at[h]],
                    nrows_v.at[pl.ds(h * 128, 128)], sem))
                cps.append(pltpu.async_copy(
                    etab_hbm.at[eidx_v.at[h]],
                    erows_v.at[pl.ds(h * 128, 128)], sem))
            for cp in cps:
                cp.wait()

            pltpu.sync_copy(nrows_v, out_hbm.at[pl.ds(base, B)])
            pltpu.sync_copy(erows_v, out_hbm.at[pl.ds(LB + base, B)])


_BR = 2048
_LR = _BR // B


def _tc_body(vmt_ref, wt_ref, pe_ref, out_ref):
    x = jax.lax.dot_general(
        wt_ref[...], vmt_ref[...],
        dimension_numbers=(((1,), (0,)), ((), ())),
        preferred_element_type=jnp.float32)
    for j in range(_LR):
        out_ref[j] = x[:, j * B:(j + 1) * B] + pe_ref[j]


def _tc_matmul(vm_t, w_t, pe3):
    return pl.pallas_call(
        _tc_body,
        grid=(LB // _BR,),
        in_specs=[
            pl.BlockSpec((NUM_VAL_TOKENS, _BR), lambda i: (0, i)),
            pl.BlockSpec((D, NUM_VAL_TOKENS), lambda i: (0, 0)),
            pl.BlockSpec((_LR, D, 1), lambda i: (i, 0, 0)),
        ],
        out_specs=pl.BlockSpec((_LR, D, B), lambda i: (2 * L // _LR + i, 0, 0)),
        out_shape=jax.ShapeDtypeStruct((3 * L, D, B), jnp.float32),
    )(vm_t, w_t, pe3)


_SB = 8


def _unpack_body(sc_ref, pe_ref, _o_ref, out_ref):
    for j in range(_SB):
        out_ref[j] = sc_ref[pl.ds(j * B, B), :].T + pe_ref[j]


def _tc_unpack(sc_out, pe3, out3):
    return pl.pallas_call(
        _unpack_body,
        grid=(2 * L // _SB,),
        in_specs=[
            pl.BlockSpec((_SB * B, D), lambda i: (i, 0)),
            pl.BlockSpec((_SB, D, 1), lambda i: (i % (L // _SB), 0, 0)),
            pl.BlockSpec(memory_space=pl.ANY),
        ],
        out_specs=pl.BlockSpec((_SB, D, B), lambda i: (i, 0, 0)),
        out_shape=jax.ShapeDtypeStruct((3 * L, D, B), jnp.float32),
        input_output_aliases={2: 0},
    )(sc_out, pe3, out3)


def kernel(node_idx, edge_idx, node_val_mat, node_embed_table, edge_embed_table,
           val_tok_embed):
    pe3 = jnp.asarray(_PE.reshape(L, D, 1))
    nidx = node_idx.astype(jnp.int32).reshape(L, 2, 128)
    eidx = edge_idx.astype(jnp.int32).reshape(L, 2, 128)
    out3 = _tc_matmul(node_val_mat.T, val_tok_embed.T, pe3)
    sc_out = _sc_gather(nidx, eidx, node_embed_table, edge_embed_table)
    out3 = _tc_unpack(sc_out, pe3, out3)
    return out3.transpose(0, 2, 1)
